# Initial kernel scaffold; baseline (speedup 1.0000x reference)
import jax
import jax.numpy as jnp
from jax.experimental import pallas as pl


def kernel(node_embeddings, node_sign_influence, adj_matrix, Wq, bq, Wk, bk, Wv, bv, Wo, bo, sign_weight):
    raise NotImplementedError("write your pallas kernel here")


# speedup vs baseline: 5636.8994x; 5636.8994x over previous
import math

import jax
import jax.numpy as jnp
from jax.experimental import pallas as pl

_N = 1024
_D = 64
_H = 2
_BLK = 256


def _attn_body(x_ref, sign_ref, adj_ref,
               wq_ref, bq_ref, wk_ref, bk_ref, wv_ref, bv_ref,
               wo_ref, bo_ref, out_ref):
    x = x_ref[:]
    xb = x_ref[pl.ds(pl.program_id(0) * _BLK, _BLK), :]
    inv_sqrt_d = 1.0 / math.sqrt(_D)

    k_all = jnp.dot(x, wk_ref[:].T, preferred_element_type=jnp.float32) + bk_ref[:]
    v_all = jnp.dot(x, wv_ref[:].T, preferred_element_type=jnp.float32) + bv_ref[:]
    q_blk = jnp.dot(xb, wq_ref[:].T, preferred_element_type=jnp.float32) + bq_ref[:]

    adj = adj_ref[:]
    mask = adj > 0.0
    sign = sign_ref[:]

    heads = []
    for h in range(_H):
        qh = q_blk[:, h * _D:(h + 1) * _D]
        kh = k_all[:, h * _D:(h + 1) * _D]
        vh = v_all[:, h * _D:(h + 1) * _D]
        s = jnp.dot(qh, kh.T, preferred_element_type=jnp.float32)
        s = s * (sign * inv_sqrt_d)
        s_masked = jnp.where(mask, s, -jnp.inf)
        m = jnp.max(s_masked, axis=1, keepdims=True)
        m = jnp.where(jnp.isfinite(m), m, 0.0)
        w = jnp.where(mask, jnp.exp(s - m), 0.0)
        denom = jnp.sum(w, axis=1, keepdims=True) + 1e-10
        heads.append(jnp.dot(w / denom, vh, preferred_element_type=jnp.float32))

    out_heads = jnp.concatenate(heads, axis=1)
    out_ref[:] = (jnp.dot(out_heads, wo_ref[:].T, preferred_element_type=jnp.float32)
                  + bo_ref[:])


def kernel(node_embeddings, node_sign_influence, adj_matrix,
           Wq, bq, Wk, bk, Wv, bv, Wo, bo, sign_weight):
    del sign_weight
    n = node_embeddings.shape[0]
    sign2d = node_sign_influence.reshape(n, 1)
    grid = (n // _BLK,)
    return pl.pallas_call(
        _attn_body,
        grid=grid,
        in_specs=[
            pl.BlockSpec((n, _D), lambda i: (0, 0)),
            pl.BlockSpec((_BLK, 1), lambda i: (i, 0)),
            pl.BlockSpec((_BLK, n), lambda i: (i, 0)),
            pl.BlockSpec((_D * _H, _D), lambda i: (0, 0)),
            pl.BlockSpec((1, _D * _H), lambda i: (0, 0)),
            pl.BlockSpec((_D * _H, _D), lambda i: (0, 0)),
            pl.BlockSpec((1, _D * _H), lambda i: (0, 0)),
            pl.BlockSpec((_D * _H, _D), lambda i: (0, 0)),
            pl.BlockSpec((1, _D * _H), lambda i: (0, 0)),
            pl.BlockSpec((_D, _D * _H), lambda i: (0, 0)),
            pl.BlockSpec((1, _D), lambda i: (0, 0)),
        ],
        out_specs=pl.BlockSpec((_BLK, _D), lambda i: (i, 0)),
        out_shape=jax.ShapeDtypeStruct((n, _D), jnp.float32),
    )(node_embeddings, sign2d, adj_matrix,
      Wq, bq.reshape(1, -1), Wk, bk.reshape(1, -1), Wv, bv.reshape(1, -1),
      Wo, bo.reshape(1, -1))
